# baseline (speedup 1.0000x reference)
import functools

import jax
import jax.numpy as jnp
from jax import lax
from jax.experimental import pallas as pl
from jax.experimental.pallas import tpu as pltpu
from jax.experimental.pallas import tpu_sc as plsc

NC = 2
NS = 16
CH = 128


def _fourier_msg(x, w2, grid_size):
    n, in_feats = x.shape
    two_gi = w2.shape[0]
    out_feats = w2.shape[1]
    bn = 400
    assert n % bn == 0

    def body(x_ref, w_ref, o_ref, feats_ref):
        xb = x_ref[...]
        c1 = jnp.cos(xb)
        s1 = jnp.sin(xb)
        ck, sk = c1, s1
        for k in range(grid_size):
            feats_ref[:, k * in_feats:(k + 1) * in_feats] = ck.astype(jnp.bfloat16)
            feats_ref[:, (grid_size + k) * in_feats:(grid_size + k + 1) * in_feats] = (
                sk.astype(jnp.bfloat16))
            if k + 1 < grid_size:
                ck, sk = ck * c1 - sk * s1, sk * c1 + ck * s1
        o_ref[...] = jnp.dot(feats_ref[...], w_ref[...],
                             preferred_element_type=jnp.float32)

    return pl.pallas_call(
        body,
        grid=(n // bn,),
        in_specs=[
            pl.BlockSpec((bn, in_feats), lambda i: (i, 0)),
            pl.BlockSpec((two_gi, out_feats), lambda i: (0, 0)),
        ],
        out_specs=pl.BlockSpec((bn, out_feats), lambda i: (i, 0)),
        out_shape=jax.ShapeDtypeStruct((n, out_feats), jnp.float32),
        scratch_shapes=[pltpu.VMEM((bn, two_gi), jnp.bfloat16)],
    )(x, w2)


def _edge_scatter(msg, src3, dst3, zeros_blk):
    n, out_feats = msg.shape
    nchunks = src3.shape[0]
    nw = NC * NS
    full_rounds = nchunks // nw
    extra = nchunks - full_rounds * nw
    npairs = full_rounds // 2
    assert full_rounds % 2 == 0
    row_stride = (n // NS) // 8 * 8
    row_win = n - (NS - 1) * row_stride
    assert row_win >= row_stride and row_win % 8 == 0

    mesh = plsc.VectorSubcoreMesh(core_axis_name="c", subcore_axis_name="s")

    blk = 26
    nblk = full_rounds // blk
    assert full_rounds == nblk * blk and blk % 2 == 0

    @functools.partial(
        pl.kernel,
        out_type=jax.ShapeDtypeStruct((NC, n, out_feats), jnp.float32),
        mesh=mesh,
        scratch_types=[
            pltpu.VMEM((blk, 1, CH), jnp.int32),
            pltpu.VMEM((blk, 1, CH), jnp.int32),
            pltpu.VMEM((blk, 1, CH), jnp.int32),
            pltpu.VMEM((blk, 1, CH), jnp.int32),
            pltpu.VMEM((2, 1, CH), jnp.int32),
            pltpu.VMEM((CH, out_feats), jnp.float32),
            pltpu.VMEM((CH, out_feats), jnp.float32),
            pltpu.VMEM_SHARED((n, out_feats), jnp.float32),
            pltpu.SemaphoreType.DMA,
            pltpu.SemaphoreType.DMA,
            pltpu.SemaphoreType.DMA,
            pltpu.SemaphoreType.DMA,
        ],
    )
    def k(msg_hbm, src_hbm, dst_hbm, zero_hbm, out_hbm,
          sb0, sb1, db0, db1, lft, rows0, rows1, acc, semi0, semi1, sem0, sem1):
        c = lax.axis_index("c")
        s = lax.axis_index("s")
        w = c * NS + s
        base = w * full_rounds

        sbufs = (sb0, sb1)
        dbufs = (db0, db1)
        sdesc = pltpu.async_copy(src_hbm.at[pl.ds(base, blk)], sb0, semi0)
        ddesc = pltpu.async_copy(dst_hbm.at[pl.ds(base, blk)], db0, semi1)
        pltpu.sync_copy(zero_hbm, acc.at[pl.ds(s * row_stride, row_win)])
        plsc.subcore_barrier()

        for ib in range(nblk):
            sb = sbufs[ib % 2]
            db = dbufs[ib % 2]
            sdesc.wait()
            ddesc.wait()
            if ib + 1 < nblk:
                nxt = pl.ds(base + (ib + 1) * blk, blk)
                sdesc = pltpu.async_copy(src_hbm.at[nxt],
                                         sbufs[(ib + 1) % 2], semi0)
                ddesc = pltpu.async_copy(dst_hbm.at[nxt],
                                         dbufs[(ib + 1) % 2], semi1)

            pltpu.async_copy(msg_hbm.at[sb.at[0].at[0]], rows0, sem0)

            @pl.loop(0, blk // 2)
            def _(p):
                a = 2 * p
                b = a + 1
                cb = pltpu.async_copy(msg_hbm.at[sb.at[b].at[0]], rows1, sem1)
                pltpu.make_async_copy(msg_hbm.at[sb.at[a].at[0]], rows0,
                                      sem0).wait()
                pltpu.sync_copy(rows0, acc.at[db.at[a].at[0]], add=True)
                @pl.when(p < blk // 2 - 1)
                def _():
                    pltpu.async_copy(msg_hbm.at[sb.at[a + 2].at[0]], rows0,
                                     sem0)
                cb.wait()
                pltpu.sync_copy(rows1, acc.at[db.at[b].at[0]], add=True)

        @pl.when(w < extra)
        def _():
            g = full_rounds * nw + w
            pltpu.sync_copy(src_hbm.at[g], lft.at[0])
            pltpu.sync_copy(dst_hbm.at[g], lft.at[1])
            pltpu.async_copy(msg_hbm.at[lft.at[0].at[0]], rows0, sem0).wait()
            pltpu.sync_copy(rows0, acc.at[lft.at[1].at[0]], add=True)

        plsc.subcore_barrier()
        pltpu.sync_copy(acc.at[pl.ds(s * row_stride, row_win)],
                        out_hbm.at[c].at[pl.ds(s * row_stride, row_win)])

    return k(msg, src3, dst3, zeros_blk)


def _combine(parts, bias2d):
    _, n, out_feats = parts.shape
    bn = 1000
    assert n % bn == 0

    def body(p_ref, b_ref, o_ref):
        o_ref[...] = p_ref[0] + p_ref[1] + b_ref[...]

    return pl.pallas_call(
        body,
        grid=(n // bn,),
        in_specs=[
            pl.BlockSpec((NC, bn, out_feats), lambda i: (0, i, 0)),
            pl.BlockSpec((1, out_feats), lambda i: (0, 0)),
        ],
        out_specs=pl.BlockSpec((bn, out_feats), lambda i: (i, 0)),
        out_shape=jax.ShapeDtypeStruct((n, out_feats), jnp.float32),
    )(parts, bias2d)


def kernel(x, edge_index, fouriercoeffs, bias):
    n, in_feats = x.shape
    out_feats = fouriercoeffs.shape[1]
    grid_size = fouriercoeffs.shape[3]
    e = edge_index.shape[1]
    assert e % CH == 0
    w2 = jnp.transpose(fouriercoeffs, (0, 3, 2, 1)).reshape(
        2 * grid_size * in_feats, out_feats).astype(jnp.bfloat16)
    msg = _fourier_msg(x, w2, grid_size)
    src3 = edge_index[0].reshape(e // CH, 1, CH)
    dst3 = edge_index[1].reshape(e // CH, 1, CH)
    row_win = n - (NS - 1) * ((n // NS) // 8 * 8)
    zeros_blk = jnp.zeros((row_win, out_feats), jnp.float32)
    parts = _edge_scatter(msg, src3, dst3, zeros_blk)
    return _combine(parts, bias.reshape(1, out_feats))


# speedup vs baseline: 1.0269x; 1.0269x over previous
import functools

import jax
import jax.numpy as jnp
from jax import lax
from jax.experimental import pallas as pl
from jax.experimental.pallas import tpu as pltpu
from jax.experimental.pallas import tpu_sc as plsc

NC = 2
NS = 16
CH = 128


def _fourier_msg(x, w2, grid_size):
    n, in_feats = x.shape
    two_gi = w2.shape[0]
    out_feats = w2.shape[1]
    bn = 1000
    assert n % bn == 0

    def body(x_ref, w_ref, o_ref, feats_ref):
        xb = x_ref[...]
        c1 = jnp.cos(xb)
        s1 = jnp.sin(xb)
        t2 = c1 + c1
        ckm, skm = c1, s1
        ck, sk = t2 * c1 - 1.0, t2 * s1
        for k in range(grid_size):
            feats_ref[:, k * in_feats:(k + 1) * in_feats] = ckm.astype(jnp.bfloat16)
            feats_ref[:, (grid_size + k) * in_feats:(grid_size + k + 1) * in_feats] = (
                skm.astype(jnp.bfloat16))
            if k + 2 < grid_size:
                ckm, skm, ck, sk = ck, sk, t2 * ck - ckm, t2 * sk - skm
            else:
                ckm, skm = ck, sk
        o_ref[...] = jnp.dot(feats_ref[...], w_ref[...],
                             preferred_element_type=jnp.float32)

    return pl.pallas_call(
        body,
        grid=(n // bn,),
        in_specs=[
            pl.BlockSpec((bn, in_feats), lambda i: (i, 0)),
            pl.BlockSpec((two_gi, out_feats), lambda i: (0, 0)),
        ],
        out_specs=pl.BlockSpec((bn, out_feats), lambda i: (i, 0)),
        out_shape=jax.ShapeDtypeStruct((n, out_feats), jnp.float32),
        scratch_shapes=[pltpu.VMEM((bn, two_gi), jnp.bfloat16)],
    )(x, w2)


def _edge_scatter(msg, src3, dst3, zeros_blk):
    n, out_feats = msg.shape
    nchunks = src3.shape[0]
    nw = NC * NS
    full_rounds = nchunks // nw
    extra = nchunks - full_rounds * nw
    npairs = full_rounds // 2
    assert full_rounds % 2 == 0
    row_stride = (n // NS) // 8 * 8
    row_win = n - (NS - 1) * row_stride
    assert row_win >= row_stride and row_win % 8 == 0

    mesh = plsc.VectorSubcoreMesh(core_axis_name="c", subcore_axis_name="s")

    blk = 26
    nblk = full_rounds // blk
    assert full_rounds == nblk * blk and blk % 2 == 0

    @functools.partial(
        pl.kernel,
        out_type=jax.ShapeDtypeStruct((NC, n, out_feats), jnp.float32),
        mesh=mesh,
        scratch_types=[
            pltpu.VMEM((blk, 1, CH), jnp.int32),
            pltpu.VMEM((blk, 1, CH), jnp.int32),
            pltpu.VMEM((blk, 1, CH), jnp.int32),
            pltpu.VMEM((blk, 1, CH), jnp.int32),
            pltpu.VMEM((2, 1, CH), jnp.int32),
            pltpu.VMEM((CH, out_feats), jnp.float32),
            pltpu.VMEM((CH, out_feats), jnp.float32),
            pltpu.VMEM_SHARED((n, out_feats), jnp.float32),
            pltpu.SemaphoreType.DMA,
            pltpu.SemaphoreType.DMA,
            pltpu.SemaphoreType.DMA,
            pltpu.SemaphoreType.DMA,
        ],
    )
    def k(msg_hbm, src_hbm, dst_hbm, zero_hbm, out_hbm,
          sb0, sb1, db0, db1, lft, rows0, rows1, acc, semi0, semi1, sem0, sem1):
        c = lax.axis_index("c")
        s = lax.axis_index("s")
        w = c * NS + s
        base = w * full_rounds

        sbufs = (sb0, sb1)
        dbufs = (db0, db1)
        sdesc = pltpu.async_copy(src_hbm.at[pl.ds(base, blk)], sb0, semi0)
        ddesc = pltpu.async_copy(dst_hbm.at[pl.ds(base, blk)], db0, semi1)
        pltpu.sync_copy(zero_hbm, acc.at[pl.ds(s * row_stride, row_win)])
        plsc.subcore_barrier()

        for ib in range(nblk):
            sb = sbufs[ib % 2]
            db = dbufs[ib % 2]
            sdesc.wait()
            ddesc.wait()
            if ib + 1 < nblk:
                nxt = pl.ds(base + (ib + 1) * blk, blk)
                sdesc = pltpu.async_copy(src_hbm.at[nxt],
                                         sbufs[(ib + 1) % 2], semi0)
                ddesc = pltpu.async_copy(dst_hbm.at[nxt],
                                         dbufs[(ib + 1) % 2], semi1)

            pltpu.async_copy(msg_hbm.at[sb.at[0].at[0]], rows0, sem0)

            @pl.loop(0, blk // 2)
            def _(p):
                a = 2 * p
                b = a + 1
                cb = pltpu.async_copy(msg_hbm.at[sb.at[b].at[0]], rows1, sem1)
                pltpu.make_async_copy(msg_hbm.at[sb.at[a].at[0]], rows0,
                                      sem0).wait()
                pltpu.sync_copy(rows0, acc.at[db.at[a].at[0]], add=True)
                @pl.when(p < blk // 2 - 1)
                def _():
                    pltpu.async_copy(msg_hbm.at[sb.at[a + 2].at[0]], rows0,
                                     sem0)
                cb.wait()
                pltpu.sync_copy(rows1, acc.at[db.at[b].at[0]], add=True)

        @pl.when(w < extra)
        def _():
            g = full_rounds * nw + w
            pltpu.sync_copy(src_hbm.at[g], lft.at[0])
            pltpu.sync_copy(dst_hbm.at[g], lft.at[1])
            pltpu.async_copy(msg_hbm.at[lft.at[0].at[0]], rows0, sem0).wait()
            pltpu.sync_copy(rows0, acc.at[lft.at[1].at[0]], add=True)

        plsc.subcore_barrier()
        pltpu.sync_copy(acc.at[pl.ds(s * row_stride, row_win)],
                        out_hbm.at[c].at[pl.ds(s * row_stride, row_win)])

    return k(msg, src3, dst3, zeros_blk)


def _combine(parts, bias2d):
    _, n, out_feats = parts.shape
    bn = 1000
    assert n % bn == 0

    def body(p_ref, b_ref, o_ref):
        o_ref[...] = p_ref[0] + p_ref[1] + b_ref[...]

    return pl.pallas_call(
        body,
        grid=(n // bn,),
        in_specs=[
            pl.BlockSpec((NC, bn, out_feats), lambda i: (0, i, 0)),
            pl.BlockSpec((1, out_feats), lambda i: (0, 0)),
        ],
        out_specs=pl.BlockSpec((bn, out_feats), lambda i: (i, 0)),
        out_shape=jax.ShapeDtypeStruct((n, out_feats), jnp.float32),
    )(parts, bias2d)


def kernel(x, edge_index, fouriercoeffs, bias):
    n, in_feats = x.shape
    out_feats = fouriercoeffs.shape[1]
    grid_size = fouriercoeffs.shape[3]
    e = edge_index.shape[1]
    assert e % CH == 0
    w2 = jnp.transpose(fouriercoeffs, (0, 3, 2, 1)).reshape(
        2 * grid_size * in_feats, out_feats).astype(jnp.bfloat16)
    msg = _fourier_msg(x, w2, grid_size)
    src3 = edge_index[0].reshape(e // CH, 1, CH)
    dst3 = edge_index[1].reshape(e // CH, 1, CH)
    row_win = n - (NS - 1) * ((n // NS) // 8 * 8)
    zeros_blk = jnp.zeros((row_win, out_feats), jnp.float32)
    parts = _edge_scatter(msg, src3, dst3, zeros_blk)
    return _combine(parts, bias.reshape(1, out_feats))


# speedup vs baseline: 1.0331x; 1.0061x over previous
import functools

import jax
import jax.numpy as jnp
from jax import lax
from jax.experimental import pallas as pl
from jax.experimental.pallas import tpu as pltpu
from jax.experimental.pallas import tpu_sc as plsc

NC = 2
NS = 16
CH = 128


def _fourier_msg(x, w2, grid_size):
    n, in_feats = x.shape
    two_gi = w2.shape[0]
    out_feats = w2.shape[1]
    bn = 1000
    assert n % bn == 0

    def body(x_ref, w_ref, o_ref, feats_ref):
        xb = x_ref[...]
        c1 = jnp.cos(xb)
        s1 = jnp.sin(xb)
        t2 = c1 + c1
        ckm, skm = c1, s1
        ck, sk = t2 * c1 - 1.0, t2 * s1
        for k in range(grid_size):
            feats_ref[:, k * in_feats:(k + 1) * in_feats] = ckm.astype(jnp.bfloat16)
            feats_ref[:, (grid_size + k) * in_feats:(grid_size + k + 1) * in_feats] = (
                skm.astype(jnp.bfloat16))
            if k + 2 < grid_size:
                ckm, skm, ck, sk = ck, sk, t2 * ck - ckm, t2 * sk - skm
            else:
                ckm, skm = ck, sk
        o_ref[...] = jnp.dot(feats_ref[...], w_ref[...],
                             preferred_element_type=jnp.float32)

    return pl.pallas_call(
        body,
        grid=(n // bn,),
        in_specs=[
            pl.BlockSpec((bn, in_feats), lambda i: (i, 0)),
            pl.BlockSpec((two_gi, out_feats), lambda i: (0, 0)),
        ],
        out_specs=pl.BlockSpec((bn, out_feats), lambda i: (i, 0)),
        out_shape=jax.ShapeDtypeStruct((n, out_feats), jnp.float32),
        scratch_shapes=[pltpu.VMEM((bn, two_gi), jnp.bfloat16)],
    )(x, w2)


def _edge_scatter(msg, src3, dst3, zeros_blk):
    n, out_feats = msg.shape
    nchunks = src3.shape[0]
    nw = NC * NS
    full_rounds = nchunks // nw
    extra = nchunks - full_rounds * nw
    npairs = full_rounds // 2
    assert full_rounds % 2 == 0
    row_stride = (n // NS) // 8 * 8
    row_win = n - (NS - 1) * row_stride
    assert row_win >= row_stride and row_win % 8 == 0

    mesh = plsc.VectorSubcoreMesh(core_axis_name="c", subcore_axis_name="s")

    blk = 26
    nblk = full_rounds // blk
    assert full_rounds == nblk * blk and blk % 2 == 0

    @functools.partial(
        pl.kernel,
        out_type=jax.ShapeDtypeStruct((NC, n, out_feats), jnp.float32),
        mesh=mesh,
        scratch_types=[
            pltpu.VMEM((blk, 1, CH), jnp.int32),
            pltpu.VMEM((blk, 1, CH), jnp.int32),
            pltpu.VMEM((blk, 1, CH), jnp.int32),
            pltpu.VMEM((blk, 1, CH), jnp.int32),
            pltpu.VMEM((2, 1, CH), jnp.int32),
            pltpu.VMEM((CH, out_feats), jnp.float32),
            pltpu.VMEM((CH, out_feats), jnp.float32),
            pltpu.VMEM_SHARED((n, out_feats), jnp.float32),
            pltpu.SemaphoreType.DMA,
            pltpu.SemaphoreType.DMA,
            pltpu.SemaphoreType.DMA,
            pltpu.SemaphoreType.DMA,
        ],
    )
    def k(msg_hbm, src_hbm, dst_hbm, zero_hbm, out_hbm,
          sb0, sb1, db0, db1, lft, rows0, rows1, acc, semi0, semi1, sem0, sem1):
        c = lax.axis_index("c")
        s = lax.axis_index("s")
        w = c * NS + s
        base = w * full_rounds

        sbufs = (sb0, sb1)
        dbufs = (db0, db1)
        sdesc = pltpu.async_copy(src_hbm.at[pl.ds(base, blk)], sb0, semi0)
        ddesc = pltpu.async_copy(dst_hbm.at[pl.ds(base, blk)], db0, semi1)
        pltpu.sync_copy(zero_hbm, acc.at[pl.ds(s * row_stride, row_win)])
        sdesc.wait()
        ddesc.wait()
        pltpu.async_copy(msg_hbm.at[sb0.at[0].at[0]], rows0, sem0)
        plsc.subcore_barrier()

        for ib in range(nblk):
            sb = sbufs[ib % 2]
            db = dbufs[ib % 2]
            if ib > 0:
                sdesc.wait()
                ddesc.wait()
            if ib + 1 < nblk:
                nxt = pl.ds(base + (ib + 1) * blk, blk)
                sdesc = pltpu.async_copy(src_hbm.at[nxt],
                                         sbufs[(ib + 1) % 2], semi0)
                ddesc = pltpu.async_copy(dst_hbm.at[nxt],
                                         dbufs[(ib + 1) % 2], semi1)

            if ib > 0:
                pltpu.async_copy(msg_hbm.at[sb.at[0].at[0]], rows0, sem0)

            @pl.loop(0, blk // 2)
            def _(p):
                a = 2 * p
                b = a + 1
                cb = pltpu.async_copy(msg_hbm.at[sb.at[b].at[0]], rows1, sem1)
                pltpu.make_async_copy(msg_hbm.at[sb.at[a].at[0]], rows0,
                                      sem0).wait()
                pltpu.sync_copy(rows0, acc.at[db.at[a].at[0]], add=True)
                @pl.when(p < blk // 2 - 1)
                def _():
                    pltpu.async_copy(msg_hbm.at[sb.at[a + 2].at[0]], rows0,
                                     sem0)
                cb.wait()
                pltpu.sync_copy(rows1, acc.at[db.at[b].at[0]], add=True)

        @pl.when(w < extra)
        def _():
            g = full_rounds * nw + w
            pltpu.sync_copy(src_hbm.at[g], lft.at[0])
            pltpu.sync_copy(dst_hbm.at[g], lft.at[1])
            pltpu.async_copy(msg_hbm.at[lft.at[0].at[0]], rows0, sem0).wait()
            pltpu.sync_copy(rows0, acc.at[lft.at[1].at[0]], add=True)

        plsc.subcore_barrier()
        pltpu.sync_copy(acc.at[pl.ds(s * row_stride, row_win)],
                        out_hbm.at[c].at[pl.ds(s * row_stride, row_win)])

    return k(msg, src3, dst3, zeros_blk)


def _combine(parts, bias2d):
    _, n, out_feats = parts.shape
    bn = 1000
    assert n % bn == 0

    def body(p_ref, b_ref, o_ref):
        o_ref[...] = p_ref[0] + p_ref[1] + b_ref[...]

    return pl.pallas_call(
        body,
        grid=(n // bn,),
        in_specs=[
            pl.BlockSpec((NC, bn, out_feats), lambda i: (0, i, 0)),
            pl.BlockSpec((1, out_feats), lambda i: (0, 0)),
        ],
        out_specs=pl.BlockSpec((bn, out_feats), lambda i: (i, 0)),
        out_shape=jax.ShapeDtypeStruct((n, out_feats), jnp.float32),
    )(parts, bias2d)


def kernel(x, edge_index, fouriercoeffs, bias):
    n, in_feats = x.shape
    out_feats = fouriercoeffs.shape[1]
    grid_size = fouriercoeffs.shape[3]
    e = edge_index.shape[1]
    assert e % CH == 0
    w2 = jnp.transpose(fouriercoeffs, (0, 3, 2, 1)).reshape(
        2 * grid_size * in_feats, out_feats).astype(jnp.bfloat16)
    msg = _fourier_msg(x, w2, grid_size)
    src3 = edge_index[0].reshape(e // CH, 1, CH)
    dst3 = edge_index[1].reshape(e // CH, 1, CH)
    row_win = n - (NS - 1) * ((n // NS) // 8 * 8)
    zeros_blk = jnp.zeros((row_win, out_feats), jnp.float32)
    parts = _edge_scatter(msg, src3, dst3, zeros_blk)
    return _combine(parts, bias.reshape(1, out_feats))
